# Initial kernel scaffold; baseline (speedup 1.0000x reference)
import jax
import jax.numpy as jnp
from jax.experimental import pallas as pl


def kernel(x, params, idx):
    raise NotImplementedError("write your pallas kernel here")


# speedup vs baseline: 1.2962x; 1.2962x over previous
import functools

import numpy as np

import jax
import jax.numpy as jnp
from jax import lax
from jax.experimental import pallas as pl
from jax.experimental.pallas import tpu as pltpu
from jax.experimental.pallas import tpu_sc as plsc

_LEVELS = [40962, 10242, 2562, 642, 162]
_EPS = 1e-5

_NC = 2
_NS = 16
_NW = _NC * _NS


def _rup(x, m):
    return (x + m - 1) // m * m


@functools.lru_cache(maxsize=None)
def _make_sc_gather(V, D, Bp):
    assert Bp % (8 * _NW) == 0
    b_per_w = Bp // _NW
    C = min(128 if D <= 256 else 64, b_per_w)
    nfull = b_per_w // C
    tail = b_per_w % C

    mesh = plsc.VectorSubcoreMesh(core_axis_name="c", subcore_axis_name="s")

    @functools.partial(
        pl.kernel,
        mesh=mesh,
        out_type=jax.ShapeDtypeStruct((Bp, D), jnp.float32),
        compiler_params=pltpu.CompilerParams(use_tc_tiling_on_sc=False),
        scratch_types=[
            pltpu.VMEM((b_per_w,), jnp.int32),
            pltpu.VMEM((C, D), jnp.float32),
            pltpu.SemaphoreType.DMA,
        ],
    )
    def gather_kernel(table_hbm, idx_hbm, out_hbm, idx_v, rows_v, sem):
        wid = lax.axis_index("s") * _NC + lax.axis_index("c")
        base = wid * b_per_w
        pltpu.sync_copy(idx_hbm.at[pl.ds(base, b_per_w)], idx_v)

        def body(c, carry):
            off = c * C
            pltpu.async_copy(
                table_hbm.at[idx_v.at[pl.ds(off, C)]], rows_v, sem
            ).wait()
            pltpu.sync_copy(rows_v, out_hbm.at[pl.ds(base + off, C)])
            return carry

        lax.fori_loop(0, nfull, body, 0)
        if tail:
            off = nfull * C
            pltpu.async_copy(
                table_hbm.at[idx_v.at[pl.ds(off, tail)]],
                rows_v.at[pl.ds(0, tail)],
                sem,
            ).wait()
            pltpu.sync_copy(
                rows_v.at[pl.ds(0, tail)], out_hbm.at[pl.ds(base + off, tail)]
            )

    return gather_kernel


def _sc_gather(table, idxp):
    V, D = table.shape
    (Bp,) = idxp.shape
    return _make_sc_gather(V, D, Bp)(table, idxp)


def _pick_bn(M):
    for b in (512, 256, 128, 64, 32, 16, 8):
        if M % b == 0:
            return b
    raise ValueError(M)


def _tc_matmul(X, W, b, nvalid=None):
    M, K = X.shape
    K2, F = W.shape
    assert K == K2
    BN = _pick_bn(M)
    grid = (M // BN,)
    stats = nvalid is not None

    def body(x_ref, w_ref, b_ref, y_ref, *rest):
        i = pl.program_id(0)
        y = (
            jnp.dot(x_ref[...], w_ref[...], preferred_element_type=jnp.float32)
            + b_ref[...]
        )
        y_ref[...] = y
        if stats:
            s_ref, ss_ref = rest
            rows = lax.broadcasted_iota(jnp.int32, (BN, 1), 0) + i * BN
            m = (rows < nvalid).astype(jnp.float32)
            ym = y * m
            ps = jnp.sum(ym, axis=0, keepdims=True)
            pss = jnp.sum(ym * ym, axis=0, keepdims=True)

            @pl.when(i == 0)
            def _():
                s_ref[...] = jnp.zeros_like(s_ref)
                ss_ref[...] = jnp.zeros_like(ss_ref)

            s_ref[...] += ps
            ss_ref[...] += pss

    in_specs = [
        pl.BlockSpec((BN, K), lambda i: (i, 0)),
        pl.BlockSpec((K, F), lambda i: (0, 0)),
        pl.BlockSpec((1, F), lambda i: (0, 0)),
    ]
    out_shapes = [jax.ShapeDtypeStruct((M, F), jnp.float32)]
    out_specs = [pl.BlockSpec((BN, F), lambda i: (i, 0))]
    if stats:
        out_shapes += [
            jax.ShapeDtypeStruct((1, F), jnp.float32),
            jax.ShapeDtypeStruct((1, F), jnp.float32),
        ]
        out_specs += [
            pl.BlockSpec((1, F), lambda i: (0, 0)),
            pl.BlockSpec((1, F), lambda i: (0, 0)),
        ]

    res = pl.pallas_call(
        body,
        grid=grid,
        in_specs=in_specs,
        out_specs=out_specs,
        out_shape=out_shapes,
    )(X, W, b.reshape(1, F))
    if stats:
        return res[0], res[1], res[2]
    return res[0]


def _tc_bn_act(y, scale, shift):
    M, F = y.shape
    BN = _pick_bn(M)

    def body(y_ref, sc_ref, sh_ref, h_ref):
        v = y_ref[...] * sc_ref[...] + sh_ref[...]
        h_ref[...] = jnp.where(v >= 0, v, 0.2 * v)

    return pl.pallas_call(
        body,
        grid=(M // BN,),
        in_specs=[
            pl.BlockSpec((BN, F), lambda i: (i, 0)),
            pl.BlockSpec((1, F), lambda i: (0, 0)),
            pl.BlockSpec((1, F), lambda i: (0, 0)),
        ],
        out_specs=pl.BlockSpec((BN, F), lambda i: (i, 0)),
        out_shape=jax.ShapeDtypeStruct((M, F), jnp.float32),
    )(y, scale, shift)


def _bn_finalize(s, ss, n, bnp):
    mu = s[0] / n
    var = jnp.maximum(ss[0] / n - mu * mu, 0.0)
    rstd = lax.rsqrt(var + _EPS)
    scale = bnp["g"] * rstd
    shift = bnp["b"] - mu * scale
    return scale.reshape(1, -1), shift.reshape(1, -1)


def _pad_idx(a, Bp):
    B = a.shape[0]
    return jnp.pad(a, (0, Bp - B)) if Bp != B else a


def _pool_matrix(f):
    P = np.zeros((7 * f, f), np.float32)
    P[np.arange(7 * f), np.arange(7 * f) // 7] = 1.0 / 7.0
    return jnp.asarray(P)


def _updown_matrix(f):
    Q = np.zeros((2 * f, f), np.float32)
    Q[np.arange(2 * f), np.arange(2 * f) // 2] = 0.5
    return jnp.asarray(Q)


def _double_conv(table, n, npad, no_pad, p):
    D = table.shape[1]
    fout = p["c1"]["W"].shape[1]
    G1 = _sc_gather(table, no_pad).reshape(npad, 7 * D)
    y1, s1, ss1 = _tc_matmul(G1, p["c1"]["W"], p["c1"]["b"], nvalid=n)
    sc1, sh1 = _bn_finalize(s1, ss1, n, p["bn1"])
    h1 = _tc_bn_act(y1, sc1, sh1)
    G2 = _sc_gather(h1, no_pad).reshape(npad, 7 * fout)
    y2, s2, ss2 = _tc_matmul(G2, p["c2"]["W"], p["c2"]["b"], nvalid=n)
    sc2, sh2 = _bn_finalize(s2, ss2, n, p["bn2"])
    return _tc_bn_act(y2, sc2, sh2)


def kernel(x, params, idx):
    levels = _LEVELS
    npads = [_rup(n, 512) for n in levels]
    no_pad = [
        _pad_idx(idx["neigh_%d" % n], npads[i] * 7)
        for i, n in enumerate(levels)
    ]

    n0 = levels[0]
    x16 = jnp.pad(x, ((0, 0), (0, 13)))
    W1 = params["down1"]["c1"]["W"].reshape(7, 3, -1)
    W1p = jnp.zeros((7, 16, W1.shape[2]), jnp.float32).at[:, :3, :].set(W1)
    W1p = W1p.reshape(7 * 16, -1)
    p1 = {
        "c1": {"W": W1p, "b": params["down1"]["c1"]["b"]},
        "bn1": params["down1"]["bn1"],
        "c2": params["down1"]["c2"],
        "bn2": params["down1"]["bn2"],
    }

    skips = []
    table = x16
    h = None
    for i in range(5):
        n, npad = levels[i], npads[i]
        p = p1 if i == 0 else params["down%d" % (i + 1)]
        h = _double_conv(table, n, npad, no_pad[i], p)
        if i < 4:
            skips.append(h)
            r, rpad = levels[i + 1], npads[i + 1]
            f = h.shape[1]
            pool_idx = _pad_idx(idx["neigh_%d" % n][: r * 7], rpad * 7)
            Gp = _sc_gather(h, pool_idx).reshape(rpad, 7 * f)
            table = _tc_matmul(Gp, _pool_matrix(f), jnp.zeros((f,), jnp.float32))

    for j in range(1, 5):
        lev = 4 - j
        n, npad = levels[lev], npads[lev]
        r = (n + 6) // 4
        p = params["up%d" % j]
        fout = p["c1"]["W"].shape[1]

        y_up = _tc_matmul(h, p["up"]["W"], p["up"]["b"])
        y_rows = y_up.reshape(-1, fout)

        top = idx["up_top_%d" % n]
        down = idx["up_down_%d" % n]
        Bp_t = _rup(top.shape[0], 8 * _NW)
        Bp_d = _rup(down.shape[0], 8 * _NW)
        x1 = _sc_gather(y_rows, _pad_idx(top, Bp_t))[:r]
        Gd = _sc_gather(y_rows, _pad_idx(down, Bp_d)).reshape(Bp_d // 2, 2 * fout)
        x2 = _tc_matmul(Gd, _updown_matrix(fout), jnp.zeros((fout,), jnp.float32))
        h_up = jnp.concatenate([x1, x2[: n - r]], axis=0)

        skip = skips[lev][:n]
        hc = jnp.concatenate([h_up, skip], axis=1)
        h = _double_conv(hc, n, npad, no_pad[lev], p)

    out = _tc_matmul(h, params["outc"]["W"], params["outc"]["b"])
    return out[: levels[0]]


# speedup vs baseline: 1.6001x; 1.2345x over previous
import functools

import numpy as np

import jax
import jax.numpy as jnp
from jax import lax
from jax.experimental import pallas as pl
from jax.experimental.pallas import tpu as pltpu
from jax.experimental.pallas import tpu_sc as plsc

_LEVELS = [40962, 10242, 2562, 642, 162]
_EPS = 1e-5

_NC = 2
_NS = 16
_NW = _NC * _NS


def _rup(x, m):
    return (x + m - 1) // m * m


@functools.lru_cache(maxsize=None)
def _make_sc_gather(V, D, Bp):
    assert Bp % (8 * _NW) == 0
    b_per_w = Bp // _NW
    C = min(128 if D <= 256 else 64, b_per_w)
    NBUF = max(1, min(4, 262144 // (C * D * 4)))
    nfull = b_per_w // C
    tail = b_per_w % C
    ngrp = nfull // NBUF
    nrem = nfull % NBUF

    mesh = plsc.VectorSubcoreMesh(core_axis_name="c", subcore_axis_name="s")
    scratch = [pltpu.VMEM((b_per_w,), jnp.int32)]
    scratch += [pltpu.VMEM((C, D), jnp.float32) for _ in range(NBUF)]
    scratch += [pltpu.SemaphoreType.DMA for _ in range(2 * NBUF)]

    @functools.partial(
        pl.kernel,
        mesh=mesh,
        out_type=jax.ShapeDtypeStruct((Bp, D), jnp.float32),
        compiler_params=pltpu.CompilerParams(use_tc_tiling_on_sc=False),
        scratch_types=scratch,
    )
    def gather_kernel(table_hbm, idx_hbm, out_hbm, idx_v, *rest):
        bufs = rest[:NBUF]
        gsem = rest[NBUF : 2 * NBUF]
        wsem = rest[2 * NBUF : 3 * NBUF]
        wid = lax.axis_index("s") * _NC + lax.axis_index("c")
        base = wid * b_per_w
        pltpu.sync_copy(idx_hbm.at[pl.ds(base, b_per_w)], idx_v)

        def wb_wait(b):
            pltpu.make_async_copy(bufs[b], out_hbm.at[pl.ds(0, C)], wsem[b]).wait()

        def group(g, carry):
            handles = []
            for b in range(NBUF):
                off = (g * NBUF + b) * C

                @pl.when(g > 0)
                def _(b=b):
                    wb_wait(b)

                handles.append(
                    (
                        pltpu.async_copy(
                            table_hbm.at[idx_v.at[pl.ds(off, C)]], bufs[b], gsem[b]
                        ),
                        off,
                    )
                )
            for b in range(NBUF):
                h, off = handles[b]
                h.wait()
                pltpu.async_copy(bufs[b], out_hbm.at[pl.ds(base + off, C)], wsem[b])
            return carry

        if ngrp > 0:
            lax.fori_loop(0, ngrp, group, 0)
            for b in range(NBUF):
                wb_wait(b)

        off0 = ngrp * NBUF * C
        for j in range(nrem):
            off = off0 + j * C
            pltpu.async_copy(
                table_hbm.at[idx_v.at[pl.ds(off, C)]], bufs[0], gsem[0]
            ).wait()
            pltpu.sync_copy(bufs[0], out_hbm.at[pl.ds(base + off, C)])
        if tail:
            off = nfull * C
            pltpu.async_copy(
                table_hbm.at[idx_v.at[pl.ds(off, tail)]],
                bufs[0].at[pl.ds(0, tail)],
                gsem[0],
            ).wait()
            pltpu.sync_copy(
                bufs[0].at[pl.ds(0, tail)], out_hbm.at[pl.ds(base + off, tail)]
            )

    return gather_kernel


def _sc_gather(table, idxp):
    V, D = table.shape
    (Bp,) = idxp.shape
    return _make_sc_gather(V, D, Bp)(table, idxp)


def _pick_bn(M):
    for b in (512, 256, 128, 64, 32, 16, 8):
        if M % b == 0:
            return b
    raise ValueError(M)


def _tc_matmul(parts, bias, nvalid=None):
    M = parts[0][0].shape[0]
    F = parts[0][1].shape[1]
    BN = _pick_bn(M)
    stats = nvalid is not None
    meta = tuple(p[2] is not None for p in parts)

    def body(*refs):
        i = pl.program_id(0)
        it = iter(refs)
        acc = None
        for has_pro in meta:
            x = next(it)[...]
            w = next(it)[...]
            if has_pro:
                sc, sh, sl = next(it)[...], next(it)[...], next(it)[...]
                v = x * sc + sh
                x = jnp.maximum(v, 0.0) + sl * jnp.minimum(v, 0.0)
            d = jnp.dot(x, w, preferred_element_type=jnp.float32)
            acc = d if acc is None else acc + d
        y = acc + next(it)[...]
        y_ref = next(it)
        y_ref[...] = y
        if stats:
            s_ref = next(it)
            ss_ref = next(it)
            rows = lax.broadcasted_iota(jnp.int32, (BN, 1), 0) + i * BN
            m = (rows < nvalid).astype(jnp.float32)
            ym = y * m
            ps = jnp.sum(ym, axis=0, keepdims=True)
            pss = jnp.sum(ym * ym, axis=0, keepdims=True)

            @pl.when(i == 0)
            def _():
                s_ref[...] = jnp.zeros_like(s_ref)
                ss_ref[...] = jnp.zeros_like(ss_ref)

            s_ref[...] += ps
            ss_ref[...] += pss

    in_specs = []
    args = []
    for X, W, pro in parts:
        K = X.shape[1]
        in_specs.append(pl.BlockSpec((BN, K), lambda i: (i, 0)))
        in_specs.append(pl.BlockSpec((K, F), lambda i: (0, 0)))
        args += [X, W]
        if pro is not None:
            for p in pro:
                in_specs.append(pl.BlockSpec((1, K), lambda i: (0, 0)))
                args.append(p)
    in_specs.append(pl.BlockSpec((1, F), lambda i: (0, 0)))
    args.append(bias.reshape(1, F))

    out_shapes = [jax.ShapeDtypeStruct((M, F), jnp.float32)]
    out_specs = [pl.BlockSpec((BN, F), lambda i: (i, 0))]
    if stats:
        out_shapes += [jax.ShapeDtypeStruct((1, F), jnp.float32)] * 2
        out_specs += [pl.BlockSpec((1, F), lambda i: (0, 0))] * 2

    res = pl.pallas_call(
        body,
        grid=(M // BN,),
        in_specs=in_specs,
        out_specs=out_specs,
        out_shape=out_shapes,
    )(*args)
    if stats:
        return res[0], res[1], res[2]
    return res[0]


def _bn_finalize(s, ss, n, bnp):
    mu = s[0] / n
    var = jnp.maximum(ss[0] / n - mu * mu, 0.0)
    rstd = lax.rsqrt(var + _EPS)
    scale = bnp["g"] * rstd
    shift = bnp["b"] - mu * scale
    slope = jnp.full_like(scale, 0.2)
    return scale, shift, slope


def _pro2d(pro, reps=1):
    return tuple(jnp.tile(p, reps).reshape(1, -1) for p in pro)


def _pad_idx(a, Bp):
    B = a.shape[0]
    return jnp.pad(a, (0, Bp - B)) if Bp != B else a


def _drop_self(no, n):
    return no.reshape(n, 7)[:, :6].reshape(-1)


def _pool_matrices(f):
    P = np.zeros((7 * f, f), np.float32)
    P[np.arange(7 * f), np.arange(7 * f) // 7] = 1.0 / 7.0
    return jnp.asarray(P[: 6 * f]), jnp.asarray(P[6 * f :])


def _updown_matrix(f):
    Q = np.zeros((2 * f, f), np.float32)
    Q[np.arange(2 * f), np.arange(2 * f) // 2] = 0.5
    return jnp.asarray(Q)


def _conv(table, no6_pad, W, b, npad, n, pro):
    f = table.shape[1]
    G6 = _sc_gather(table, no6_pad).reshape(npad, 6 * f)
    parts = [
        (G6, W[: 6 * f], _pro2d(pro, 6) if pro is not None else None),
        (table, W[6 * f :], _pro2d(pro) if pro is not None else None),
    ]
    return _tc_matmul(parts, b, nvalid=n)


def _double_conv(table, n, npad, no6_pad, p, pro_in):
    y1, s1, ss1 = _conv(table, no6_pad, p["c1"]["W"], p["c1"]["b"], npad, n, pro_in)
    pro1 = _bn_finalize(s1, ss1, n, p["bn1"])
    y2, s2, ss2 = _conv(y1, no6_pad, p["c2"]["W"], p["c2"]["b"], npad, n, pro1)
    pro2 = _bn_finalize(s2, ss2, n, p["bn2"])
    return y2, pro2


def kernel(x, params, idx):
    levels = _LEVELS
    npads = [_rup(n, 512) for n in levels]
    no6_pad = []
    for i, n in enumerate(levels):
        no6 = _drop_self(idx["neigh_%d" % n], n)
        no6_pad.append(_pad_idx(no6, npads[i] * 6))

    x16 = jnp.pad(x, ((0, npads[0] - levels[0]), (0, 13)))
    W1 = params["down1"]["c1"]["W"].reshape(7, 3, -1)
    W1p = jnp.zeros((7, 16, W1.shape[2]), jnp.float32).at[:, :3, :].set(W1)
    W1p = W1p.reshape(7 * 16, -1)
    p1 = {
        "c1": {"W": W1p, "b": params["down1"]["c1"]["b"]},
        "bn1": params["down1"]["bn1"],
        "c2": params["down1"]["c2"],
        "bn2": params["down1"]["bn2"],
    }

    skips = []
    table, pro_in = x16, None
    for i in range(5):
        n, npad = levels[i], npads[i]
        p = p1 if i == 0 else params["down%d" % (i + 1)]
        y, pro = _double_conv(table, n, npad, no6_pad[i], p, pro_in)
        if i < 4:
            skips.append((y, pro))
            r, rpad = levels[i + 1], npads[i + 1]
            f = y.shape[1]
            pno6 = _pad_idx(
                _drop_self(idx["neigh_%d" % n][: r * 7], r), rpad * 6
            )
            Gp = _sc_gather(y, pno6).reshape(rpad, 6 * f)
            Pa, Pb = _pool_matrices(f)
            table = _tc_matmul(
                [(Gp, Pa, _pro2d(pro, 6)), (y[:rpad], Pb, _pro2d(pro))],
                jnp.zeros((f,), jnp.float32),
            )
            pro_in = None
        else:
            h, pro_h = y, pro

    for j in range(1, 5):
        lev = 4 - j
        n, npad = levels[lev], npads[lev]
        r = (n + 6) // 4
        p = params["up%d" % j]
        fout = p["c1"]["W"].shape[1]

        y_up = _tc_matmul(
            [(h, p["up"]["W"], _pro2d(pro_h))], p["up"]["b"]
        )
        y_rows = y_up.reshape(-1, fout)

        top = idx["up_top_%d" % n]
        down = idx["up_down_%d" % n]
        Bp_t = _rup(top.shape[0], 8 * _NW)
        Bp_d = _rup(down.shape[0], 8 * _NW)
        x1 = _sc_gather(y_rows, _pad_idx(top, Bp_t))[:r]
        Gd = _sc_gather(y_rows, _pad_idx(down, Bp_d)).reshape(Bp_d // 2, 2 * fout)
        x2 = _tc_matmul(
            [(Gd, _updown_matrix(fout), None)], jnp.zeros((fout,), jnp.float32)
        )
        h_up = jnp.concatenate([x1, x2[: n - r]], axis=0)

        y_skip, pro_skip = skips[lev]
        hc = jnp.concatenate([h_up, y_skip[:n]], axis=1)
        hc = jnp.pad(hc, ((0, npad - n), (0, 0)))
        ones = jnp.ones((fout,), jnp.float32)
        zeros = jnp.zeros((fout,), jnp.float32)
        pro_c = (
            jnp.concatenate([ones, pro_skip[0]]),
            jnp.concatenate([zeros, pro_skip[1]]),
            jnp.concatenate([ones, pro_skip[2]]),
        )
        h, pro_h = _double_conv(hc, n, npad, no6_pad[lev], p, pro_c)

    out = _tc_matmul(
        [(h, params["outc"]["W"], _pro2d(pro_h))], params["outc"]["b"]
    )
    return out[: levels[0]]
